# baseline (speedup 1.0000x reference)
import functools

import jax
import jax.numpy as jnp
from jax import lax
from jax.experimental import pallas as pl
from jax.experimental.pallas import tpu as pltpu
from jax.experimental.pallas import tpu_sc as plsc

_DIM = 64
_SCALE = 8.0
_NC = 2
_NS = 16
_NW = _NC * _NS
_S = 128
_NB = 4
_B = 4096 * 200
_NSTEP = _B // (_NW * _S)


def _body(x_hbm, tbl_hbm, out_hbm, idx_v,
          rin0, rin1, rin2, rin3, rout0, rout1, rout2, rout3,
          g0, g1, g2, g3, s0, s1, s2, s3):
  rins = [rin0, rin1, rin2, rin3]
  routs = [rout0, rout1, rout2, rout3]
  gsems = [g0, g1, g2, g3]
  ssems = [s0, s1, s2, s3]

  wid = lax.axis_index("s") * _NC + lax.axis_index("c")
  row0 = wid * _NSTEP
  base = wid * (_NSTEP * _S)

  pltpu.sync_copy(x_hbm.at[pl.ds(row0, _NSTEP)], idx_v)

  def wait_gather(b):
    pltpu.make_async_copy(tbl_hbm.at[idx_v.at[0]], rins[b], gsems[b]).wait()

  def wait_scatter(b):
    pltpu.make_async_copy(routs[b], out_hbm.at[pl.ds(0, _S)], ssems[b]).wait()

  for b in range(_NB):
    pltpu.async_copy(tbl_hbm.at[idx_v.at[b]], rins[b], gsems[b])

  def outer(o, carry):
    for b in range(_NB):
      g = o * _NB + b
      wait_gather(b)

      @pl.when(g >= _NB)
      def _():
        wait_scatter(b)

      def scale_row(i, c):
        for l in range(_DIM // 16):
          routs[b][i, pl.ds(l * 16, 16)] = (
              rins[b][i, pl.ds(l * 16, 16)] * _SCALE)
        return c
      lax.fori_loop(0, _S, scale_row, 0, unroll=2)

      @pl.when(g < _NSTEP - _NB)
      def _():
        pltpu.async_copy(tbl_hbm.at[idx_v.at[g + _NB]], rins[b], gsems[b])

      pltpu.async_copy(routs[b], out_hbm.at[pl.ds(base + g * _S, _S)],
                       ssems[b])
    return carry

  lax.fori_loop(0, _NSTEP // _NB, outer, 0)

  for b in range(_NB):
    wait_scatter(b)


_sc_call = pl.kernel(
    _body,
    out_type=jax.ShapeDtypeStruct((_B, _DIM), jnp.float32),
    mesh=plsc.VectorSubcoreMesh(core_axis_name="c", subcore_axis_name="s"),
    scratch_types=(
        [pltpu.VMEM((_NSTEP, _S), jnp.int32)]
        + [pltpu.VMEM((_S, _DIM), jnp.float32) for _ in range(2 * _NB)]
        + [pltpu.SemaphoreType.DMA for _ in range(2 * _NB)]
    ),
    compiler_params=pltpu.CompilerParams(use_tc_tiling_on_sc=False),
)


@jax.jit
def kernel(x, table):
  xi = x.astype(jnp.int32).reshape(_B // _S, _S)
  out = _sc_call(xi, table)
  return out.reshape(4096, 200, _DIM)


# speedup vs baseline: 1.2683x; 1.2683x over previous
import functools

import jax
import jax.numpy as jnp
from jax import lax
from jax.experimental import pallas as pl
from jax.experimental.pallas import tpu as pltpu
from jax.experimental.pallas import tpu_sc as plsc

_DIM = 64
_SCALE = 8.0
_NC = 2
_NS = 16
_NW = _NC * _NS
_S = 128
_NB = 4
_B = 4096 * 200
_NSTEP = _B // (_NW * _S)


def _body(x_hbm, tbl_hbm, out_hbm, idx_v,
          rin0, rin1, rin2, rin3, rout0, rout1, rout2, rout3,
          g0, g1, g2, g3, s0, s1, s2, s3):
  rins = [rin0, rin1, rin2, rin3]
  routs = [rout0, rout1, rout2, rout3]
  gsems = [g0, g1, g2, g3]
  ssems = [s0, s1, s2, s3]

  wid = lax.axis_index("s") * _NC + lax.axis_index("c")
  row0 = wid * _NSTEP
  base = wid * (_NSTEP * _S)

  pltpu.sync_copy(x_hbm.at[pl.ds(row0, _NSTEP)], idx_v)

  def wait_gather(b):
    pltpu.make_async_copy(tbl_hbm.at[idx_v.at[0]], rins[b], gsems[b]).wait()

  def wait_scatter(b):
    pltpu.make_async_copy(routs[b], out_hbm.at[pl.ds(0, _S)], ssems[b]).wait()

  for b in range(_NB):
    pltpu.async_copy(tbl_hbm.at[idx_v.at[b]], rins[b], gsems[b])

  def outer(o, carry):
    for b in range(_NB):
      g = o * _NB + b
      wait_gather(b)

      @pl.when(g >= _NB)
      def _():
        wait_scatter(b)

      @plsc.parallel_loop(0, _S, unroll=4)
      def _(i):
        for l in range(_DIM // 16):
          routs[b][i, pl.ds(l * 16, 16)] = (
              rins[b][i, pl.ds(l * 16, 16)] * _SCALE)

      @pl.when(g < _NSTEP - _NB)
      def _():
        pltpu.async_copy(tbl_hbm.at[idx_v.at[g + _NB]], rins[b], gsems[b])

      pltpu.async_copy(routs[b], out_hbm.at[pl.ds(base + g * _S, _S)],
                       ssems[b])
    return carry

  lax.fori_loop(0, _NSTEP // _NB, outer, 0)

  for b in range(_NB):
    wait_scatter(b)


_sc_call = pl.kernel(
    _body,
    out_type=jax.ShapeDtypeStruct((_B, _DIM), jnp.float32),
    mesh=plsc.VectorSubcoreMesh(core_axis_name="c", subcore_axis_name="s"),
    scratch_types=(
        [pltpu.VMEM((_NSTEP, _S), jnp.int32)]
        + [pltpu.VMEM((_S, _DIM), jnp.float32) for _ in range(2 * _NB)]
        + [pltpu.SemaphoreType.DMA for _ in range(2 * _NB)]
    ),
    compiler_params=pltpu.CompilerParams(use_tc_tiling_on_sc=False),
)


@jax.jit
def kernel(x, table):
  xi = x.astype(jnp.int32).reshape(_B // _S, _S)
  out = _sc_call(xi, table)
  return out.reshape(4096, 200, _DIM)
